# Initial kernel scaffold; baseline (speedup 1.0000x reference)
import functools
import math

import jax
import jax.numpy as jnp
from jax import lax
from jax.experimental import pallas as pl
from jax.experimental.pallas import tpu as pltpu
from jax.experimental.pallas import tpu_sc as plsc

D_MODEL = 64
_SCALE = math.sqrt(D_MODEL)

_NC = 2
_NS = 16
_NW = _NC * _NS

_CHUNK = 128
_NBUF = 4


def _scale_body(t_ref, o_ref):
    o_ref[...] = t_ref[...] * _SCALE


def _scale_table(table):
    v, d = table.shape
    blk = 2000
    assert v % blk == 0
    return pl.pallas_call(
        _scale_body,
        out_shape=jax.ShapeDtypeStruct((v, d), table.dtype),
        grid=(v // blk,),
        in_specs=[pl.BlockSpec((blk, d), lambda i: (i, 0))],
        out_specs=pl.BlockSpec((blk, d), lambda i: (i, 0)),
    )(table)


def _make_gather(num_idx, d):
    assert num_idx % (_NW * _CHUNK) == 0
    cpw = num_idx // (_NW * _CHUNK)
    groups = cpw // _NBUF
    assert cpw % _NBUF == 0

    mesh = plsc.VectorSubcoreMesh(core_axis_name="c", subcore_axis_name="s")

    @functools.partial(
        pl.kernel,
        out_type=jax.ShapeDtypeStruct((num_idx, d), jnp.float32),
        mesh=mesh,
        scratch_types=(
            [pltpu.VMEM((cpw, _CHUNK), jnp.int32)]
            + [pltpu.VMEM((_CHUNK, d), jnp.float32) for _ in range(_NBUF)]
            + [pltpu.SemaphoreType.DMA for _ in range(_NBUF)]
        ),
    )
    def gather(idx_hbm, tab_hbm, out_hbm, idx_v, *rest):
        rows = rest[:_NBUF]
        sems = rest[_NBUF:]
        wid = lax.axis_index("s") * _NC + lax.axis_index("c")
        pltpu.sync_copy(idx_hbm.at[pl.ds(wid * cpw, cpw)], idx_v)

        def fire(t, b):
            pltpu.async_copy(tab_hbm.at[idx_v.at[t]], rows[b], sems[b])

        def wait(b):
            pltpu.make_async_copy(
                tab_hbm.at[idx_v.at[0]], rows[b], sems[b]
            ).wait()

        def put(t, b):
            off = (wid * cpw + t) * _CHUNK
            pltpu.sync_copy(rows[b], out_hbm.at[pl.ds(off, _CHUNK)])

        for b in range(_NBUF):
            fire(b, b)

        def body(g, carry):
            for b in range(_NBUF):
                t = g * _NBUF + b
                wait(b)
                put(t, b)
                fire(t + _NBUF, b)
            return carry

        lax.fori_loop(0, groups - 1, body, 0)

        for b in range(_NBUF):
            t = (groups - 1) * _NBUF + b
            wait(b)
            put(t, b)

    return gather


def kernel(x, table):
    bsz, seq = x.shape
    v, d = table.shape
    num_idx = bsz * seq
    scaled = _scale_table(table)
    idx2 = x.reshape(num_idx // _CHUNK, _CHUNK)
    out = _make_gather(num_idx, d)(idx2, scaled)
    return out.reshape(bsz, seq, d)


# speedup vs baseline: 3.8740x; 3.8740x over previous
import functools
import math

import jax
import jax.numpy as jnp
from jax import lax
from jax.experimental import pallas as pl
from jax.experimental.pallas import tpu as pltpu
from jax.experimental.pallas import tpu_sc as plsc

D_MODEL = 64
_SCALE = math.sqrt(D_MODEL)

_NC = 2
_NS = 16
_NW = _NC * _NS

_CHUNK = 128
_NBUF = 4


def _scale_body(t_ref, o_ref):
    o_ref[...] = t_ref[...] * _SCALE


def _scale_table(table):
    v, d = table.shape
    blk = 2000
    assert v % blk == 0
    return pl.pallas_call(
        _scale_body,
        out_shape=jax.ShapeDtypeStruct((v, d), table.dtype),
        grid=(v // blk,),
        in_specs=[pl.BlockSpec((blk, d), lambda i: (i, 0))],
        out_specs=pl.BlockSpec((blk, d), lambda i: (i, 0)),
    )(table)


def _make_gather(num_idx, d):
    assert num_idx % (_NW * _CHUNK) == 0
    cpw = num_idx // (_NW * _CHUNK)
    groups = cpw // _NBUF
    assert cpw % _NBUF == 0

    mesh = plsc.VectorSubcoreMesh(core_axis_name="c", subcore_axis_name="s")

    @functools.partial(
        pl.kernel,
        out_type=jax.ShapeDtypeStruct((num_idx, d), jnp.float32),
        mesh=mesh,
        compiler_params=pltpu.CompilerParams(use_tc_tiling_on_sc=False),
        scratch_types=(
            [pltpu.VMEM((cpw, _CHUNK), jnp.int32)]
            + [pltpu.VMEM((_CHUNK, d), jnp.float32) for _ in range(_NBUF)]
            + [pltpu.SemaphoreType.DMA for _ in range(_NBUF)]
        ),
    )
    def gather(idx_hbm, tab_hbm, out_hbm, idx_v, *rest):
        rows = rest[:_NBUF]
        sems = rest[_NBUF:]
        wid = lax.axis_index("s") * _NC + lax.axis_index("c")
        pltpu.sync_copy(idx_hbm.at[pl.ds(wid * cpw, cpw)], idx_v)

        def fire(t, b):
            pltpu.async_copy(tab_hbm.at[idx_v.at[t]], rows[b], sems[b])

        def wait(b):
            pltpu.make_async_copy(
                tab_hbm.at[idx_v.at[0]], rows[b], sems[b]
            ).wait()

        def put(t, b):
            off = (wid * cpw + t) * _CHUNK
            pltpu.sync_copy(rows[b], out_hbm.at[pl.ds(off, _CHUNK)])

        for b in range(_NBUF):
            fire(b, b)

        def body(g, carry):
            for b in range(_NBUF):
                t = g * _NBUF + b
                wait(b)
                put(t, b)
                fire(t + _NBUF, b)
            return carry

        lax.fori_loop(0, groups - 1, body, 0)

        for b in range(_NBUF):
            t = (groups - 1) * _NBUF + b
            wait(b)
            put(t, b)

    return gather


def kernel(x, table):
    bsz, seq = x.shape
    v, d = table.shape
    num_idx = bsz * seq
    scaled = _scale_table(table)
    idx2 = x.reshape(num_idx // _CHUNK, _CHUNK)
    out = _make_gather(num_idx, d)(idx2, scaled)
    return out.reshape(bsz, seq, d)


# speedup vs baseline: 7.0211x; 1.8123x over previous
import functools
import math

import jax
import jax.numpy as jnp
from jax import lax
from jax.experimental import pallas as pl
from jax.experimental.pallas import tpu as pltpu
from jax.experimental.pallas import tpu_sc as plsc

D_MODEL = 64
_SCALE = math.sqrt(D_MODEL)

_NC = 2
_NS = 16
_NW = _NC * _NS

_CHUNK = 128
_SC = 2
_NBUF = 2
_L = 16


def _make_gather(seq, nb, d):
    nchunks = seq * nb
    assert nchunks % (_NW * _SC) == 0
    cpw = nchunks // _NW
    upw = cpw // _SC
    groups = upw // _NBUF
    assert upw % _NBUF == 0
    dh = d // 8
    bw = _SC * _CHUNK
    tile2 = _SC * 8 * _CHUNK

    mesh = plsc.VectorSubcoreMesh(core_axis_name="c", subcore_axis_name="s")

    @functools.partial(
        pl.kernel,
        out_type=jax.ShapeDtypeStruct(
            (seq * dh * nb // _SC, tile2), jnp.float32
        ),
        mesh=mesh,
        compiler_params=pltpu.CompilerParams(
            use_tc_tiling_on_sc=False, needs_layout_passes=False
        ),
        scratch_types=(
            [pltpu.VMEM((cpw, _CHUNK), jnp.int32)]
            + [pltpu.VMEM((bw, d), jnp.float32) for _ in range(_NBUF)]
            + [pltpu.VMEM((d * bw,), jnp.float32) for _ in range(_NBUF)]
            + [pltpu.SemaphoreType.DMA for _ in range(_NBUF)]
            + [pltpu.SemaphoreType.DMA for _ in range(_NBUF)]
        ),
    )
    def gather(idx_hbm, tab_hbm, out_hbm, idx_v, *rest):
        rows = rest[:_NBUF]
        tiles = rest[_NBUF:2 * _NBUF]
        gsems = rest[2 * _NBUF:3 * _NBUF]
        osems = rest[3 * _NBUF:]
        wid = lax.axis_index("s") * _NC + lax.axis_index("c")
        base = wid * upw
        pltpu.sync_copy(idx_hbm.at[pl.ds(wid * cpw, cpw)], idx_v)

        def fire(u, b):
            for k in range(_SC):
                pltpu.async_copy(
                    tab_hbm.at[idx_v.at[u * _SC + k]],
                    rows[b].at[pl.ds(k * _CHUNK, _CHUNK)],
                    gsems[b],
                )

        def gwait(b):
            pltpu.make_async_copy(
                tab_hbm.at[idx_v.at[0]], rows[b], gsems[b]
            ).wait()

        nbh2 = nb // _SC

        def put(u, b):
            c = base + u
            s = c // nbh2
            bh2 = c % nbh2
            row0 = (s * dh) * nbh2 + bh2
            for dhi in range(dh):
                pltpu.async_copy(
                    tiles[b].at[pl.ds(dhi * tile2, tile2)],
                    out_hbm.at[row0 + dhi * nbh2],
                    osems[b],
                )

        def owait(b):
            for dhi in range(dh):
                pltpu.make_async_copy(
                    tiles[b].at[pl.ds(dhi * tile2, tile2)],
                    out_hbm.at[0],
                    osems[b],
                ).wait()

        lane = lax.iota(jnp.int32, _L)
        col_q = [lane + q * _L for q in range(d // _L)]
        dvec_q = [
            ((lane + q * _L) // 8) * (_SC * 1024)
            + lax.rem(lane + q * _L, 8) * _CHUNK
            for q in range(d // _L)
        ]
        off_bg = [(bg // 8) * 1024 + (bg % 8) * _L for bg in range(bw // _L)]

        def transpose(b):
            a = rows[b]
            bb = tiles[b]

            @plsc.parallel_loop(0, _L, unroll=2)
            def _(r):
                rot = lax.rem(r + lane, _L)
                for q in range(d // _L):
                    for bg in range(bw // _L):
                        vec = plsc.load_gather(
                            a, [rot + bg * _L, col_q[q]]
                        )
                        plsc.store_scatter(
                            bb, [dvec_q[q] + (off_bg[bg] + rot)],
                            vec * _SCALE,
                        )

        for b in range(_NBUF):
            fire(b, b)

        def loop_body(g, carry):
            for b in range(_NBUF):
                u = g * _NBUF + b
                gwait(b)

                @pl.when(g > 0)
                def _():
                    owait(b)

                transpose(b)
                put(u, b)

                @pl.when(g < groups - 1)
                def _():
                    fire(u + _NBUF, b)

            return carry

        lax.fori_loop(0, groups, loop_body, 0)

        for b in range(_NBUF):
            owait(b)

    return gather


def kernel(x, table):
    bsz, seq = x.shape
    v, d = table.shape
    nb = bsz // _CHUNK
    xt = x.T.reshape(seq * nb, _CHUNK)
    o = _make_gather(seq, nb, d)(xt, table)
    o6 = o.reshape(seq, d // 8, nb // _SC, _SC, 8, _CHUNK)
    return o6.transpose(2, 3, 5, 0, 1, 4).reshape(bsz, seq, d)


# speedup vs baseline: 7.4512x; 1.0613x over previous
import functools
import math

import jax
import jax.numpy as jnp
from jax import lax
from jax.experimental import pallas as pl
from jax.experimental.pallas import tpu as pltpu
from jax.experimental.pallas import tpu_sc as plsc

D_MODEL = 64
_SCALE = math.sqrt(D_MODEL)

_NC = 2
_NS = 16
_NW = _NC * _NS

_CHUNK = 128
_NBUF = 5
_L = 16


def _make_gather(seq, nb, d):
    nchunks = seq * nb
    assert nchunks % _NW == 0
    cpw = nchunks // _NW
    groups = cpw // _NBUF
    assert cpw % _NBUF == 0
    dh = d // 8

    mesh = plsc.VectorSubcoreMesh(core_axis_name="c", subcore_axis_name="s")

    @functools.partial(
        pl.kernel,
        out_type=jax.ShapeDtypeStruct((seq * dh * nb, 8 * _CHUNK), jnp.float32),
        mesh=mesh,
        compiler_params=pltpu.CompilerParams(
            use_tc_tiling_on_sc=False, needs_layout_passes=False
        ),
        scratch_types=(
            [pltpu.VMEM((cpw, _CHUNK), jnp.int32)]
            + [pltpu.VMEM((_CHUNK, d), jnp.float32) for _ in range(_NBUF)]
            + [pltpu.VMEM((d * _CHUNK,), jnp.float32) for _ in range(_NBUF)]
            + [pltpu.SemaphoreType.DMA for _ in range(_NBUF)]
            + [pltpu.SemaphoreType.DMA for _ in range(_NBUF)]
        ),
    )
    def gather(idx_hbm, tab_hbm, out_hbm, idx_v, *rest):
        rows = rest[:_NBUF]
        tiles = rest[_NBUF:2 * _NBUF]
        gsems = rest[2 * _NBUF:3 * _NBUF]
        osems = rest[3 * _NBUF:]
        wid = lax.axis_index("s") * _NC + lax.axis_index("c")
        base = wid * cpw
        pltpu.sync_copy(idx_hbm.at[pl.ds(base, cpw)], idx_v)

        def fire(t, b):
            pltpu.async_copy(tab_hbm.at[idx_v.at[t]], rows[b], gsems[b])

        def gwait(b):
            pltpu.make_async_copy(
                tab_hbm.at[idx_v.at[0]], rows[b], gsems[b]
            ).wait()

        tile_w = 8 * _CHUNK

        def put(t, b):
            c = base + t
            s = c // nb
            bh = c % nb
            row0 = (s * dh) * nb + bh
            for dhi in range(dh):
                pltpu.async_copy(
                    tiles[b].at[pl.ds(dhi * tile_w, tile_w)],
                    out_hbm.at[row0 + dhi * nb],
                    osems[b],
                )

        def owait(b):
            for dhi in range(dh):
                pltpu.make_async_copy(
                    tiles[b].at[pl.ds(dhi * tile_w, tile_w)],
                    out_hbm.at[0],
                    osems[b],
                ).wait()

        lane = lax.iota(jnp.int32, _L)
        col_q = [lane + q * _L for q in range(d // _L)]
        dst_q = [(lane + q * _L) * _CHUNK for q in range(d // _L)]

        def transpose(b):
            a = rows[b]
            bb = tiles[b]

            @plsc.parallel_loop(0, _L, unroll=4)
            def _(r):
                rot = lax.rem(r + lane, _L)
                for q in range(d // _L):
                    for bg in range(_CHUNK // _L):
                        row = rot + bg * _L
                        vec = plsc.load_gather(a, [row, col_q[q]])
                        plsc.store_scatter(
                            bb, [dst_q[q] + row], vec * _SCALE
                        )

        for b in range(_NBUF):
            fire(b, b)

        def loop_body(g, carry):
            for b in range(_NBUF):
                t = g * _NBUF + b
                gwait(b)
                @pl.when(g > 0)
                def _():
                    owait(b)

                transpose(b)
                put(t, b)

                @pl.when(g < groups - 1)
                def _():
                    fire(t + _NBUF, b)

            return carry

        lax.fori_loop(0, groups, loop_body, 0)

        for b in range(_NBUF):
            owait(b)

    return gather


def kernel(x, table):
    bsz, seq = x.shape
    v, d = table.shape
    nb = bsz // _CHUNK
    xt = x.T.reshape(seq * nb, _CHUNK)
    o = _make_gather(seq, nb, d)(xt, table)
    o5 = o.reshape(seq, d // 8, nb, 8, _CHUNK)
    return o5.transpose(2, 4, 0, 1, 3).reshape(bsz, seq, d)


# speedup vs baseline: 7.8238x; 1.0500x over previous
import functools
import math

import jax
import jax.numpy as jnp
from jax import lax
from jax.experimental import pallas as pl
from jax.experimental.pallas import tpu as pltpu
from jax.experimental.pallas import tpu_sc as plsc

D_MODEL = 64
_SCALE = math.sqrt(D_MODEL)

_NC = 2
_NS = 16
_NW = _NC * _NS

_CHUNK = 128
_NBUF = 4
_L = 16


def _make_gather(seq, nb, d):
    nchunks = seq * nb
    assert nchunks % _NW == 0
    cpw = nchunks // _NW
    groups = cpw // _NBUF
    assert cpw % _NBUF == 0
    dh = d // 8

    mesh = plsc.VectorSubcoreMesh(core_axis_name="c", subcore_axis_name="s")

    @functools.partial(
        pl.kernel,
        out_type=jax.ShapeDtypeStruct((seq * dh * nb * 8 * _CHUNK,), jnp.float32),
        mesh=mesh,
        compiler_params=pltpu.CompilerParams(
            use_tc_tiling_on_sc=False, needs_layout_passes=False
        ),
        scratch_types=(
            [pltpu.VMEM((cpw, _CHUNK), jnp.int32)]
            + [pltpu.VMEM((_CHUNK, d), jnp.float32) for _ in range(_NBUF)]
            + [pltpu.VMEM((d * _CHUNK,), jnp.float32) for _ in range(_NBUF)]
            + [pltpu.SemaphoreType.DMA for _ in range(_NBUF)]
            + [pltpu.SemaphoreType.DMA for _ in range(_NBUF)]
        ),
    )
    def gather(idx_hbm, tab_hbm, out_hbm, idx_v, *rest):
        rows = rest[:_NBUF]
        tiles = rest[_NBUF:2 * _NBUF]
        gsems = rest[2 * _NBUF:3 * _NBUF]
        osems = rest[3 * _NBUF:]
        wid = lax.axis_index("s") * _NC + lax.axis_index("c")
        base = wid * cpw
        pltpu.sync_copy(idx_hbm.at[pl.ds(base, cpw)], idx_v)

        def fire(t, b):
            pltpu.async_copy(tab_hbm.at[idx_v.at[t]], rows[b], gsems[b])

        def gwait(b):
            pltpu.make_async_copy(
                tab_hbm.at[idx_v.at[0]], rows[b], gsems[b]
            ).wait()

        tile_w = 8 * _CHUNK

        def put(t, b):
            c = base + t
            s = c // nb
            bh = c % nb
            row0 = (s * dh) * nb + bh
            for dhi in range(dh):
                pltpu.async_copy(
                    tiles[b].at[pl.ds(dhi * tile_w, tile_w)],
                    out_hbm.at[pl.ds((row0 + dhi * nb) * tile_w, tile_w)],
                    osems[b],
                )

        def owait(b):
            pltpu.make_async_copy(
                tiles[b], out_hbm.at[pl.ds(0, d * _CHUNK)], osems[b]
            ).wait()

        lane = lax.iota(jnp.int32, _L)
        col_q = [lane + q * _L for q in range(d // _L)]
        dst_q = [(lane + q * _L) * _CHUNK for q in range(d // _L)]

        def transpose(b):
            a = rows[b]
            bb = tiles[b]

            @plsc.parallel_loop(0, _L, unroll=2)
            def _(r):
                rot = lax.rem(r + lane, _L)
                for q in range(d // _L):
                    for bg in range(_CHUNK // _L):
                        row = rot + bg * _L
                        vec = plsc.load_gather(a, [row, col_q[q]])
                        plsc.store_scatter(
                            bb, [dst_q[q] + row], vec * _SCALE
                        )

        for b in range(_NBUF):
            fire(b, b)

        def loop_body(g, carry):
            for b in range(_NBUF):
                t = g * _NBUF + b
                gwait(b)
                @pl.when(g > 0)
                def _():
                    owait(b)

                transpose(b)
                put(t, b)

                @pl.when(g < groups - 1)
                def _():
                    fire(t + _NBUF, b)

            return carry

        lax.fori_loop(0, groups, loop_body, 0)

        for b in range(_NBUF):
            owait(b)

    return gather


def kernel(x, table):
    bsz, seq = x.shape
    v, d = table.shape
    nb = bsz // _CHUNK
    xt = x.T.reshape(seq * nb, _CHUNK)
    o = _make_gather(seq, nb, d)(xt, table)
    o5 = o.reshape(seq, d // 8, nb, 8, _CHUNK)
    return o5.transpose(2, 4, 0, 1, 3).reshape(bsz, seq, d)


# speedup vs baseline: 8.7796x; 1.1222x over previous
import functools
import math

import jax
import jax.numpy as jnp
from jax import lax
from jax.experimental import pallas as pl
from jax.experimental.pallas import tpu as pltpu
from jax.experimental.pallas import tpu_sc as plsc

D_MODEL = 64
_SCALE = math.sqrt(D_MODEL)

_NC = 2
_NS = 16
_NW = _NC * _NS

_CHUNK = 128
_NBUF = 4
_L = 16


def _make_gather(seq, nb, d):
    nchunks = seq * nb
    assert nchunks % _NW == 0
    cpw = nchunks // _NW
    groups = cpw // _NBUF
    assert cpw % _NBUF == 0
    dh = d // 8

    mesh = plsc.VectorSubcoreMesh(core_axis_name="c", subcore_axis_name="s")

    @functools.partial(
        pl.kernel,
        out_type=jax.ShapeDtypeStruct((seq * dh * nb * 8 * _CHUNK,), jnp.float32),
        mesh=mesh,
        compiler_params=pltpu.CompilerParams(
            use_tc_tiling_on_sc=False, needs_layout_passes=False
        ),
        scratch_types=(
            [pltpu.VMEM((cpw, _CHUNK), jnp.int32)]
            + [pltpu.VMEM((_CHUNK, d), jnp.float32) for _ in range(_NBUF)]
            + [pltpu.VMEM((d * _CHUNK,), jnp.float32) for _ in range(_NBUF)]
            + [pltpu.SemaphoreType.DMA for _ in range(_NBUF)]
            + [pltpu.SemaphoreType.DMA for _ in range(_NBUF)]
        ),
    )
    def gather(idx_hbm, tab_hbm, out_hbm, idx_v, *rest):
        rows = rest[:_NBUF]
        tiles = rest[_NBUF:2 * _NBUF]
        gsems = rest[2 * _NBUF:3 * _NBUF]
        osems = rest[3 * _NBUF:]
        wid = lax.axis_index("s") * _NC + lax.axis_index("c")
        base = wid * cpw
        pltpu.sync_copy(idx_hbm.at[pl.ds(base, cpw)], idx_v)

        def fire(t, b):
            pltpu.async_copy(tab_hbm.at[idx_v.at[t]], rows[b], gsems[b])

        def gwait(b):
            pltpu.make_async_copy(
                tab_hbm.at[idx_v.at[0]], rows[b], gsems[b]
            ).wait()

        tile_w = 8 * _CHUNK

        def put(t, b, dhis):
            c = base + t
            s = c // nb
            bh = c % nb
            row0 = (s * dh) * nb + bh
            for dhi in dhis:
                pltpu.async_copy(
                    tiles[b].at[pl.ds(dhi * tile_w, tile_w)],
                    out_hbm.at[pl.ds((row0 + dhi * nb) * tile_w, tile_w)],
                    osems[b],
                )

        def owait(b):
            pltpu.make_async_copy(
                tiles[b], out_hbm.at[pl.ds(0, d * _CHUNK)], osems[b]
            ).wait()

        lane = lax.iota(jnp.int32, _L)
        col_q = [lane + q * _L for q in range(d // _L)]
        dst_q = [(lane + q * _L) * _CHUNK for q in range(d // _L)]

        def transpose(b, qs):
            a = rows[b]
            bb = tiles[b]

            @plsc.parallel_loop(0, _L, unroll=2)
            def _(r):
                rot = lax.rem(r + lane, _L)
                for q in qs:
                    for bg in range(_CHUNK // _L):
                        row = rot + bg * _L
                        vec = plsc.load_gather(a, [row, col_q[q]])
                        plsc.store_scatter(
                            bb, [dst_q[q] + row], vec * _SCALE
                        )

        for b in range(_NBUF):
            fire(b, b)

        def loop_body(g, carry):
            for b in range(_NBUF):
                t = g * _NBUF + b
                gwait(b)
                @pl.when(g > 0)
                def _():
                    owait(b)

                transpose(b, (0, 1))
                put(t, b, range(dh // 2))
                transpose(b, (2, 3))
                put(t, b, range(dh // 2, dh))

                @pl.when(g < groups - 1)
                def _():
                    fire(t + _NBUF, b)

            return carry

        lax.fori_loop(0, groups, loop_body, 0)

        for b in range(_NBUF):
            owait(b)

    return gather


def kernel(x, table):
    bsz, seq = x.shape
    v, d = table.shape
    nb = bsz // _CHUNK
    xt = x.T.reshape(seq * nb, _CHUNK)
    o = _make_gather(seq, nb, d)(xt, table)
    o5 = o.reshape(seq, d // 8, nb, 8, _CHUNK)
    return o5.transpose(2, 4, 0, 1, 3).reshape(bsz, seq, d)
